# Initial kernel scaffold; baseline (speedup 1.0000x reference)
import jax
import jax.numpy as jnp
from jax.experimental import pallas as pl
from jax.experimental.pallas import tpu as pltpu

_T = 4096
_D = 128
_V = 76
_BT = 512
_NB = _T // _BT


def _tc_body(scale_ref, ids_ref, q6_ref, table_ref, basis_ref, ham_ref, comb_ref):
    i = pl.program_id(0)
    j = pl.program_id(1)
    rows = q6_ref[pl.ds(i * _BT, _BT), :]
    cols = q6_ref[pl.ds(j * _BT, _BT), :]
    g = jax.lax.dot_general(rows, cols, (((1,), (1,)), ((), ())),
                            preferred_element_type=jnp.float32)
    si = jnp.sum(rows, axis=1)
    sj = jnp.sum(cols, axis=1)
    scale = scale_ref[0]
    ham_ref[...] = (-scale) * (si[:, None] + sj[None, :] - 2.0 * g)

    @pl.when(j == 0)
    def _():
        ids = ids_ref[pl.ds(i * _BT, _BT)]
        onehot = (ids[:, None] == jax.lax.broadcasted_iota(
            jnp.int32, (_BT, _V), 1)).astype(jnp.float32)
        std = jnp.dot(onehot, table_ref[...], preferred_element_type=jnp.float32)
        geo = jnp.dot(rows, basis_ref[...], preferred_element_type=jnp.float32)
        comb_ref[...] = std + geo


def kernel(token_ids, q6_vecs, token_embed, q6_basis, hamming_scale):
    scale = jnp.reshape(hamming_scale, (1,)).astype(jnp.float32)
    ids = token_ids.astype(jnp.int32)
    ham, comb = pl.pallas_call(
        _tc_body,
        grid=(_NB, _NB),
        in_specs=[
            pl.BlockSpec(memory_space=pltpu.SMEM),
            pl.BlockSpec(memory_space=pltpu.ANY),
            pl.BlockSpec(memory_space=pltpu.ANY),
            pl.BlockSpec(memory_space=pltpu.ANY),
            pl.BlockSpec(memory_space=pltpu.ANY),
        ],
        out_specs=[
            pl.BlockSpec((_BT, _BT), lambda i, j: (i, j)),
            pl.BlockSpec((_BT, _D), lambda i, j: (i, 0)),
        ],
        out_shape=[
            jax.ShapeDtypeStruct((_T, _T), jnp.float32),
            jax.ShapeDtypeStruct((_T, _D), jnp.float32),
        ],
        compiler_params=pltpu.CompilerParams(
            dimension_semantics=("arbitrary", "arbitrary")),
    )(scale, ids, q6_vecs, token_embed, q6_basis)
    return comb[None], ham


# speedup vs baseline: 2.2593x; 2.2593x over previous
import jax
import jax.numpy as jnp
from jax.experimental import pallas as pl
from jax.experimental.pallas import tpu as pltpu

_T = 4096
_D = 128
_V = 76
_BT = 512
_NB = _T // _BT


def _tc_body(scale_ref, ids_ref, q6_ref, table_ref, basis_ref, ham_ref, comb_ref):
    i = pl.program_id(0)
    j = pl.program_id(1)
    rows = q6_ref[pl.ds(i * _BT, _BT), :]
    cols = q6_ref[pl.ds(j * _BT, _BT), :]
    g = jax.lax.dot_general(rows, cols, (((1,), (1,)), ((), ())),
                            preferred_element_type=jnp.float32)
    si = jnp.sum(rows, axis=1)
    sj = jnp.sum(cols, axis=1)
    scale = scale_ref[0]
    ham_ref[...] = (-scale) * (si[:, None] + sj[None, :] - 2.0 * g)

    @pl.when(j == 0)
    def _():
        ids = ids_ref[pl.ds(i * _BT, _BT)]
        onehot = (ids[:, None] == jax.lax.broadcasted_iota(
            jnp.int32, (_BT, _V), 1)).astype(jnp.float32)
        std = jnp.dot(onehot, table_ref[...], preferred_element_type=jnp.float32)
        geo = jnp.dot(rows, basis_ref[...], preferred_element_type=jnp.float32)
        comb_ref[...] = std + geo


def kernel(token_ids, q6_vecs, token_embed, q6_basis, hamming_scale):
    scale = jnp.reshape(hamming_scale, (1,)).astype(jnp.float32)
    ids = token_ids.astype(jnp.int32)
    ham, comb = pl.pallas_call(
        _tc_body,
        grid=(_NB, _NB),
        in_specs=[
            pl.BlockSpec(memory_space=pltpu.SMEM),
            pl.BlockSpec(memory_space=pltpu.VMEM),
            pl.BlockSpec(memory_space=pltpu.VMEM),
            pl.BlockSpec(memory_space=pltpu.VMEM),
            pl.BlockSpec(memory_space=pltpu.VMEM),
        ],
        out_specs=[
            pl.BlockSpec((_BT, _BT), lambda i, j: (i, j)),
            pl.BlockSpec((_BT, _D), lambda i, j: (i, 0)),
        ],
        out_shape=[
            jax.ShapeDtypeStruct((_T, _T), jnp.float32),
            jax.ShapeDtypeStruct((_T, _D), jnp.float32),
        ],
        compiler_params=pltpu.CompilerParams(
            dimension_semantics=("arbitrary", "arbitrary")),
    )(scale, ids, q6_vecs, token_embed, q6_basis)
    return comb[None], ham
